# baseline (speedup 1.0000x reference)
import functools

import jax
import jax.numpy as jnp
from jax import lax
from jax.experimental import pallas as pl
from jax.experimental.pallas import tpu as pltpu
from jax.experimental.pallas import tpu_sc as plsc

VOCAB = 100000
EMBED_DIM = 64
HIDDEN = 128
BATCH = 1024


_TILE_G = 8192


def _g_body(tt_ref, w1_ref, b1_ref, g_ref):
    acc = lax.dot_general(
        tt_ref[...], w1_ref[...], (((0,), (0,)), ((), ())),
        preferred_element_type=jnp.float32,
    )
    g_ref[...] = jnp.maximum(acc + b1_ref[...], 0.0)


def _g_kernel(tableT, W1, b1_2d):
    grid = (pl.cdiv(VOCAB, _TILE_G),)
    return pl.pallas_call(
        _g_body,
        grid=grid,
        in_specs=[
            pl.BlockSpec((EMBED_DIM, _TILE_G), lambda i: (0, i)),
            pl.BlockSpec((EMBED_DIM, HIDDEN), lambda i: (0, 0)),
            pl.BlockSpec((1, HIDDEN), lambda i: (0, 0)),
        ],
        out_specs=pl.BlockSpec((_TILE_G, HIDDEN), lambda i: (i, 0)),
        out_shape=jax.ShapeDtypeStruct((VOCAB, HIDDEN), jnp.float32),
        compiler_params=pltpu.CompilerParams(
            dimension_semantics=("arbitrary",),
        ),
    )(tableT, W1, b1_2d)


_NC = 2
_NS = 16
_NW = _NC * _NS
_B_PER_W = BATCH // _NW


@functools.cache
def _sc_gather_fn():
    mesh = plsc.VectorSubcoreMesh(core_axis_name="c", subcore_axis_name="s")

    @functools.partial(
        pl.kernel,
        mesh=mesh,
        out_type=jax.ShapeDtypeStruct((BATCH, HIDDEN), jnp.float32),
        scratch_types=[
            pltpu.VMEM((_B_PER_W,), jnp.int32),
            pltpu.VMEM((_B_PER_W, HIDDEN), jnp.float32),
            pltpu.SemaphoreType.DMA,
        ],
    )
    def _sc_gather(table_hbm, idx_hbm, out_hbm, idx_v, rows_v, sem):
        wid = lax.axis_index("s") * _NC + lax.axis_index("c")
        base = wid * _B_PER_W
        pltpu.sync_copy(idx_hbm.at[pl.ds(base, _B_PER_W)], idx_v)
        pltpu.async_copy(table_hbm.at[idx_v], rows_v, sem).wait()
        pltpu.sync_copy(rows_v, out_hbm.at[pl.ds(base, _B_PER_W)])

    return _sc_gather


_TILE_V = 4096


def _mlp_body(h_ref, w2t_ref, b2_ref, out_ref):
    acc = lax.dot_general(
        w2t_ref[...], h_ref[...], (((1,), (1,)), ((), ())),
        preferred_element_type=jnp.float32,
    )
    out_ref[...] = acc + b2_ref[...].T


def _mlp(h, W2t, b2_2d):
    grid = (pl.cdiv(VOCAB, _TILE_V),)
    return pl.pallas_call(
        _mlp_body,
        grid=grid,
        in_specs=[
            pl.BlockSpec((BATCH, HIDDEN), lambda i: (0, 0)),
            pl.BlockSpec((_TILE_V, HIDDEN), lambda i: (i, 0)),
            pl.BlockSpec((1, _TILE_V), lambda i: (0, i)),
        ],
        out_specs=pl.BlockSpec((_TILE_V, BATCH), lambda i: (i, 0)),
        out_shape=jax.ShapeDtypeStruct((VOCAB, BATCH), jnp.float32),
        compiler_params=pltpu.CompilerParams(
            dimension_semantics=("arbitrary",),
        ),
    )(h, W2t, b2_2d)


def kernel(word_idx, emb_table, W1, b1, W2, b2):
    tableT = emb_table.T
    W2t = W2.T
    G = _g_kernel(tableT, W1, b1.reshape(1, HIDDEN))
    h = jnp.take(G, word_idx, axis=0)
    out_t = _mlp(h, W2t, b2.reshape(1, VOCAB))
    return out_t.T


# speedup vs baseline: 1.0312x; 1.0312x over previous
import functools

import jax
import jax.numpy as jnp
from jax import lax
from jax.experimental import pallas as pl
from jax.experimental.pallas import tpu as pltpu
from jax.experimental.pallas import tpu_sc as plsc

VOCAB = 100000
EMBED_DIM = 64
HIDDEN = 128
BATCH = 1024


_TILE_G = 8192


def _g_body(tt_ref, w1_ref, b1_ref, g_ref):
    acc = lax.dot_general(
        tt_ref[...], w1_ref[...], (((0,), (0,)), ((), ())),
        preferred_element_type=jnp.float32,
    )
    g_ref[...] = jnp.maximum(acc + b1_ref[...], 0.0)


def _g_kernel(tableT, W1, b1_2d):
    grid = (pl.cdiv(VOCAB, _TILE_G),)
    return pl.pallas_call(
        _g_body,
        grid=grid,
        in_specs=[
            pl.BlockSpec((EMBED_DIM, _TILE_G), lambda i: (0, i)),
            pl.BlockSpec((EMBED_DIM, HIDDEN), lambda i: (0, 0)),
            pl.BlockSpec((1, HIDDEN), lambda i: (0, 0)),
        ],
        out_specs=pl.BlockSpec((_TILE_G, HIDDEN), lambda i: (i, 0)),
        out_shape=jax.ShapeDtypeStruct((VOCAB, HIDDEN), jnp.float32),
        compiler_params=pltpu.CompilerParams(
            dimension_semantics=("arbitrary",),
        ),
    )(tableT, W1, b1_2d)


_NC = 1
_NS = 16
_NW = _NC * _NS
_B_PER_W = BATCH // _NW


@functools.cache
def _sc_gather_fn():
    mesh = plsc.VectorSubcoreMesh(core_axis_name="c", subcore_axis_name="s", num_cores=1)

    @functools.partial(
        pl.kernel,
        mesh=mesh,
        out_type=jax.ShapeDtypeStruct((BATCH, HIDDEN), jnp.float32),
        scratch_types=[
            pltpu.VMEM((_B_PER_W,), jnp.int32),
            pltpu.VMEM((_B_PER_W, HIDDEN), jnp.float32),
            pltpu.SemaphoreType.DMA,
        ],
    )
    def _sc_gather(table_hbm, idx_hbm, out_hbm, idx_v, rows_v, sem):
        wid = lax.axis_index("s") * _NC + lax.axis_index("c")
        base = wid * _B_PER_W
        pltpu.sync_copy(idx_hbm.at[pl.ds(base, _B_PER_W)], idx_v)
        pltpu.async_copy(table_hbm.at[idx_v], rows_v, sem).wait()
        pltpu.sync_copy(rows_v, out_hbm.at[pl.ds(base, _B_PER_W)])

    return _sc_gather


_TILE_V = 4096


def _mlp_body(h_ref, w2t_ref, b2_ref, out_ref):
    acc = lax.dot_general(
        w2t_ref[...], h_ref[...], (((1,), (1,)), ((), ())),
        preferred_element_type=jnp.float32,
    )
    out_ref[...] = acc + b2_ref[...].T


def _mlp(h, W2t, b2_2d):
    grid = (pl.cdiv(VOCAB, _TILE_V),)
    return pl.pallas_call(
        _mlp_body,
        grid=grid,
        in_specs=[
            pl.BlockSpec((BATCH, HIDDEN), lambda i: (0, 0)),
            pl.BlockSpec((_TILE_V, HIDDEN), lambda i: (i, 0)),
            pl.BlockSpec((1, _TILE_V), lambda i: (0, i)),
        ],
        out_specs=pl.BlockSpec((_TILE_V, BATCH), lambda i: (i, 0)),
        out_shape=jax.ShapeDtypeStruct((VOCAB, BATCH), jnp.float32),
        compiler_params=pltpu.CompilerParams(
            dimension_semantics=("arbitrary",),
        ),
    )(h, W2t, b2_2d)


def kernel(word_idx, emb_table, W1, b1, W2, b2):
    tableT = emb_table.T
    W2t = W2.T
    G = _g_kernel(tableT, W1, b1.reshape(1, HIDDEN))
    h = _sc_gather_fn()(G, word_idx.astype(jnp.int32))
    out_t = _mlp(h, W2t, b2.reshape(1, VOCAB))
    return out_t.T


# speedup vs baseline: 1.0467x; 1.0150x over previous
import functools

import jax
import jax.numpy as jnp
from jax import lax
from jax.experimental import pallas as pl
from jax.experimental.pallas import tpu as pltpu
from jax.experimental.pallas import tpu_sc as plsc

VOCAB = 100000
EMBED_DIM = 64
HIDDEN = 128
BATCH = 1024


_TILE_G = 16384


def _g_body(tt_ref, w1_ref, b1_ref, g_ref):
    acc = lax.dot_general(
        tt_ref[...], w1_ref[...], (((0,), (0,)), ((), ())),
        preferred_element_type=jnp.float32,
    )
    g_ref[...] = jnp.maximum(acc + b1_ref[...], 0.0)


def _g_kernel(tableT, W1, b1_2d):
    grid = (pl.cdiv(VOCAB, _TILE_G),)
    return pl.pallas_call(
        _g_body,
        grid=grid,
        in_specs=[
            pl.BlockSpec((EMBED_DIM, _TILE_G), lambda i: (0, i)),
            pl.BlockSpec((EMBED_DIM, HIDDEN), lambda i: (0, 0)),
            pl.BlockSpec((1, HIDDEN), lambda i: (0, 0)),
        ],
        out_specs=pl.BlockSpec((_TILE_G, HIDDEN), lambda i: (i, 0)),
        out_shape=jax.ShapeDtypeStruct((VOCAB, HIDDEN), jnp.float32),
        compiler_params=pltpu.CompilerParams(
            dimension_semantics=("arbitrary",),
        ),
    )(tableT, W1, b1_2d)


_NC = 1
_NS = 16
_NW = _NC * _NS
_B_PER_W = BATCH // _NW


@functools.cache
def _sc_gather_fn():
    mesh = plsc.VectorSubcoreMesh(core_axis_name="c", subcore_axis_name="s", num_cores=1)

    @functools.partial(
        pl.kernel,
        mesh=mesh,
        out_type=jax.ShapeDtypeStruct((BATCH, HIDDEN), jnp.float32),
        scratch_types=[
            pltpu.VMEM((_B_PER_W,), jnp.int32),
            pltpu.VMEM((_B_PER_W, HIDDEN), jnp.float32),
            pltpu.SemaphoreType.DMA,
        ],
    )
    def _sc_gather(table_hbm, idx_hbm, out_hbm, idx_v, rows_v, sem):
        wid = lax.axis_index("s") * _NC + lax.axis_index("c")
        base = wid * _B_PER_W
        pltpu.sync_copy(idx_hbm.at[pl.ds(base, _B_PER_W)], idx_v)
        pltpu.async_copy(table_hbm.at[idx_v], rows_v, sem).wait()
        pltpu.sync_copy(rows_v, out_hbm.at[pl.ds(base, _B_PER_W)])

    return _sc_gather


_TILE_V = 4096


def _mlp_body(h_ref, w2t_ref, b2_ref, out_ref):
    acc = lax.dot_general(
        w2t_ref[...], h_ref[...], (((1,), (1,)), ((), ())),
        preferred_element_type=jnp.float32,
    )
    out_ref[...] = acc + b2_ref[...].T


def _mlp(h, W2t, b2_2d):
    grid = (pl.cdiv(VOCAB, _TILE_V),)
    return pl.pallas_call(
        _mlp_body,
        grid=grid,
        in_specs=[
            pl.BlockSpec((BATCH, HIDDEN), lambda i: (0, 0)),
            pl.BlockSpec((_TILE_V, HIDDEN), lambda i: (i, 0)),
            pl.BlockSpec((1, _TILE_V), lambda i: (0, i)),
        ],
        out_specs=pl.BlockSpec((_TILE_V, BATCH), lambda i: (i, 0)),
        out_shape=jax.ShapeDtypeStruct((VOCAB, BATCH), jnp.float32),
        compiler_params=pltpu.CompilerParams(
            dimension_semantics=("arbitrary",),
        ),
    )(h, W2t, b2_2d)


def kernel(word_idx, emb_table, W1, b1, W2, b2):
    tableT = emb_table.T
    W2t = W2.T
    G = _g_kernel(tableT, W1, b1.reshape(1, HIDDEN))
    h = _sc_gather_fn()(G, word_idx.astype(jnp.int32))
    out_t = _mlp(h, W2t, b2.reshape(1, VOCAB))
    return out_t.T
